# Initial kernel scaffold; baseline (speedup 1.0000x reference)
import jax
import jax.numpy as jnp
from jax.experimental import pallas as pl


def kernel(x, edge_index, edge_weight, Wz, bz, Wr, br, Wh, bh, W_lin, b_lin):
    raise NotImplementedError("write your pallas kernel here")


# speedup vs baseline: 37.1546x; 37.1546x over previous
import functools

import jax
import jax.numpy as jnp
from jax import lax
from jax.experimental import pallas as pl
from jax.experimental.pallas import tpu as pltpu
from jax.experimental.pallas import tpu_sc as plsc

N = 10000
E = 320000
F = 128
NT = 12

NC = 2
NS = 16
CHUNK = 128
RPT = 160
EP = NS * RPT * CHUNK
NP = 10240
RPN = NP // NS

_mesh = plsc.VectorSubcoreMesh(core_axis_name="c", subcore_axis_name="s")


@functools.partial(
    pl.kernel,
    out_type=jax.ShapeDtypeStruct((NC, NP), jnp.float32),
    mesh=_mesh,
    scratch_types=[
        pltpu.VMEM((RPT, CHUNK), jnp.int32),
        pltpu.VMEM((RPT, CHUNK), jnp.float32),
        pltpu.VMEM((RPN,), jnp.float32),
        pltpu.VMEM_SHARED((NP,), jnp.float32),
    ],
)
def _sc_degrees(ei_hbm, w_hbm, deg_hbm, idx_v, w_v, buf_v, acc_sh):
    c = lax.axis_index("c")
    s = lax.axis_index("s")

    pltpu.sync_copy(ei_hbm.at[c, pl.ds(s * RPT, RPT), :], idx_v)
    pltpu.sync_copy(w_hbm.at[pl.ds(s * RPT, RPT), :], w_v)

    def _z(i, _):
        buf_v[pl.ds(i * 16, 16)] = jnp.zeros((16,), jnp.float32)
        return _
    lax.fori_loop(0, RPN // 16, _z, 0)
    pltpu.sync_copy(buf_v, acc_sh.at[pl.ds(s * RPN, RPN)])
    plsc.subcore_barrier()

    def _body(k, _):
        pltpu.sync_copy(w_v.at[k], acc_sh.at[idx_v.at[k]], add=True)
        return _
    lax.fori_loop(0, RPT, _body, 0)
    plsc.subcore_barrier()

    pltpu.sync_copy(acc_sh.at[pl.ds(s * RPN, RPN)], buf_v)
    pltpu.sync_copy(buf_v, deg_hbm.at[c, pl.ds(s * RPN, RPN)])


def _table_body(x_ref, dego_ref, out_ref):
    xb = x_ref[...]
    d = dego_ref[...]
    scale = jnp.where(d > 0.0, 1.0 / d, 0.0)
    out_ref[0] = xb
    out_ref[1] = xb * scale


def _build_tables(x_pad, dego):
    nb = 10
    bs = NP // nb
    return pl.pallas_call(
        _table_body,
        grid=(nb,),
        in_specs=[
            pl.BlockSpec((bs, F), lambda i: (i, 0)),
            pl.BlockSpec((bs, 1), lambda i: (i, 0)),
        ],
        out_specs=pl.BlockSpec((2, bs, F), lambda i: (0, i, 0)),
        out_shape=jax.ShapeDtypeStruct((2, NP, F), jnp.float32),
    )(x_pad, dego)


BPB = 32


@functools.partial(
    pl.kernel,
    out_type=jax.ShapeDtypeStruct((NC, NP, F), jnp.float32),
    mesh=_mesh,
    scratch_types=[
        pltpu.VMEM((BPB, CHUNK), jnp.int32),
        pltpu.VMEM((BPB, CHUNK), jnp.int32),
        pltpu.VMEM((2, CHUNK, F), jnp.float32),
        pltpu.VMEM_SHARED((NP, F), jnp.float32),
        pltpu.SemaphoreType.DMA,
        pltpu.SemaphoreType.DMA,
    ],
)
def _sc_edge_pass(tab_hbm, ri_hbm, ci_hbm, out_hbm, r_v, c_v, g_v, acc_sh,
                  sem0, sem1):
    c = lax.axis_index("c")
    s = lax.axis_index("s")

    def _z(i, _):
        g_v[0, i // 8, pl.ds((i % 8) * 16, 16)] = jnp.zeros((16,), jnp.float32)
        return _
    lax.fori_loop(0, CHUNK * F // 16, _z, 0)
    for j in range(RPN // CHUNK):
        pltpu.sync_copy(g_v.at[0], acc_sh.at[pl.ds(s * RPN + j * CHUNK, CHUNK), :])
    plsc.subcore_barrier()

    sems = (sem0, sem1)

    def _start(k, b):
        pltpu.async_copy(tab_hbm.at[r_v.at[k]], g_v.at[b], sems[b])

    def _wait(k, b):
        pltpu.make_async_copy(tab_hbm.at[r_v.at[k]], g_v.at[b], sems[b]).wait()

    def _block(blk, carry):
        base = s * RPT + blk * BPB
        pltpu.sync_copy(ri_hbm.at[c, pl.ds(base, BPB), :], r_v)
        pltpu.sync_copy(ci_hbm.at[pl.ds(base, BPB), :], c_v)
        _start(0, 0)
        _start(1, 1)

        def _body(g, inner):
            for b in range(2):
                k = g * 2 + b
                _wait(k, b)
                pltpu.sync_copy(g_v.at[b], acc_sh.at[c_v.at[k]], add=True)

                @pl.when(k + 2 < BPB)
                def _go(b=b, k=k):
                    _start(k + 2, b)
            return inner

        lax.fori_loop(0, BPB // 2, _body, 0)
        return carry

    lax.fori_loop(0, RPT // BPB, _block, 0)
    plsc.subcore_barrier()

    for j in range(RPN // CHUNK):
        r0 = s * RPN + j * CHUNK
        pltpu.sync_copy(acc_sh.at[pl.ds(r0, CHUNK), :], g_v.at[0])
        pltpu.sync_copy(g_v.at[0], out_hbm.at[c, pl.ds(r0, CHUNK), :])


def _dense_body(x_ref, sums_ref, din_ref,
                wz00_ref, wz10_ref, wz01_ref, wz11_ref,
                wh00_ref, wh10_ref, wh01_ref, wh11_ref,
                bz_ref, bh_ref, wl_ref, bl_ref, out_ref):
    xb = x_ref[...]
    S = sums_ref[0]
    O = sums_ref[1]
    din = din_ref[...]
    Ti = S * jnp.where(din > 0.0, 1.0 / din, 0.0)

    dot = functools.partial(jnp.dot, preferred_element_type=jnp.float32)
    zp = (dot(xb, wz00_ref[...] + wz10_ref[...]) + dot(O, wz01_ref[...])
          + dot(Ti, wz11_ref[...]) + bz_ref[...])
    hp = (dot(xb, wh00_ref[...] + wh10_ref[...]) + dot(O, wh01_ref[...])
          + dot(Ti, wh11_ref[...]) + bh_ref[...])
    z = jax.nn.sigmoid(zp)
    ht = jnp.tanh(hp)
    h = jax.nn.relu((1.0 - z) * ht)
    out_ref[...] = dot(h, wl_ref[...]) + bl_ref[...]


def _dense(x_pad, sums, din, Wz, bz, Wh, bh, W_lin, b_lin):
    nb = 10
    bs = NP // nb
    full = lambda shape: pl.BlockSpec(shape, lambda i: tuple(0 for _ in shape))
    return pl.pallas_call(
        _dense_body,
        grid=(nb,),
        in_specs=[
            pl.BlockSpec((bs, F), lambda i: (i, 0)),
            pl.BlockSpec((2, bs, F), lambda i: (0, i, 0)),
            pl.BlockSpec((bs, 1), lambda i: (i, 0)),
            full((F, F)), full((F, F)), full((F, F)), full((F, F)),
            full((F, F)), full((F, F)), full((F, F)), full((F, F)),
            full((1, F)), full((1, F)), full((F, NT)), full((1, NT)),
        ],
        out_specs=pl.BlockSpec((bs, NT), lambda i: (i, 0)),
        out_shape=jax.ShapeDtypeStruct((NP, NT), jnp.float32),
    )(x_pad, sums, din,
      Wz[0, 0, :F], Wz[1, 0, :F], Wz[0, 1, :F], Wz[1, 1, :F],
      Wh[0, 0, :F], Wh[1, 0, :F], Wh[0, 1, :F], Wh[1, 1, :F],
      bz.reshape(1, F), bh.reshape(1, F), W_lin, b_lin.reshape(1, NT))


def kernel(x, edge_index, edge_weight, Wz, bz, Wr, br, Wh, bh, W_lin, b_lin):
    row = edge_index[0].astype(jnp.int32)
    col = edge_index[1].astype(jnp.int32)
    w = edge_weight.astype(jnp.float32)

    npad = EP - E
    pad_idx = (N + (jnp.arange(npad, dtype=jnp.int32) % (NP - N)))
    row_p = jnp.concatenate([row, pad_idx])
    col_p = jnp.concatenate([col, pad_idx])
    w_p = jnp.concatenate([w, jnp.zeros((npad,), jnp.float32)])

    nrows = EP // CHUNK
    ei2 = jnp.stack([row_p, col_p]).reshape(NC, nrows, CHUNK)
    w2 = w_p.reshape(nrows, CHUNK)

    degs = _sc_degrees(ei2, w2)

    x_pad = jnp.concatenate(
        [x.astype(jnp.float32), jnp.zeros((NP - N, F), jnp.float32)])
    tables = _build_tables(x_pad, degs[0].reshape(NP, 1))
    tab_flat = tables.reshape(NC * NP, F)

    ri2 = jnp.stack([row_p, row_p + NP]).reshape(NC, nrows, CHUNK)
    ci2 = col_p.reshape(nrows, CHUNK)
    sums = _sc_edge_pass(tab_flat, ri2, ci2)

    out = _dense(x_pad, sums, degs[1].reshape(NP, 1),
                 Wz, bz, Wh, bh, W_lin, b_lin)
    return out[:N]


# speedup vs baseline: 38.8584x; 1.0459x over previous
import functools

import jax
import jax.numpy as jnp
from jax import lax
from jax.experimental import pallas as pl
from jax.experimental.pallas import tpu as pltpu
from jax.experimental.pallas import tpu_sc as plsc

N = 10000
E = 320000
F = 128
NT = 12

NC = 2
NS = 16
CHUNK = 128
RPT = 160
EP = NS * RPT * CHUNK
NP = 10240
RPN = NP // NS

_mesh = plsc.VectorSubcoreMesh(core_axis_name="c", subcore_axis_name="s")


@functools.partial(
    pl.kernel,
    out_type=jax.ShapeDtypeStruct((NC, NP), jnp.float32),
    mesh=_mesh,
    scratch_types=[
        pltpu.VMEM((RPT, CHUNK), jnp.int32),
        pltpu.VMEM((RPT, CHUNK), jnp.float32),
        pltpu.VMEM((RPN,), jnp.float32),
        pltpu.VMEM_SHARED((NP,), jnp.float32),
        pltpu.SemaphoreType.DMA,
    ],
)
def _sc_degrees(ei_hbm, w_hbm, deg_hbm, idx_v, w_v, buf_v, acc_sh, sem):
    c = lax.axis_index("c")
    s = lax.axis_index("s")

    pltpu.sync_copy(ei_hbm.at[c, pl.ds(s * RPT, RPT), :], idx_v)
    pltpu.sync_copy(w_hbm.at[pl.ds(s * RPT, RPT), :], w_v)

    def _z(i, _):
        buf_v[pl.ds(i * 16, 16)] = jnp.zeros((16,), jnp.float32)
        return _
    lax.fori_loop(0, RPN // 16, _z, 0)
    pltpu.sync_copy(buf_v, acc_sh.at[pl.ds(s * RPN, RPN)])
    plsc.subcore_barrier()

    def _sst(k):
        pltpu.async_copy(w_v.at[k], acc_sh.at[idx_v.at[k]], sem, add=True)

    def _fire(k, carry):
        _sst(k)
        return carry
    lax.fori_loop(0, 32, _fire, 0)

    def _body(k, carry):
        pltpu.make_async_copy(w_v.at[k], acc_sh.at[idx_v.at[k]], sem).wait()

        @pl.when(k + 32 < RPT)
        def _go():
            _sst(k + 32)
        return carry
    lax.fori_loop(0, RPT, _body, 0)
    plsc.subcore_barrier()

    pltpu.sync_copy(acc_sh.at[pl.ds(s * RPN, RPN)], buf_v)
    pltpu.sync_copy(buf_v, deg_hbm.at[c, pl.ds(s * RPN, RPN)])


def _table_body(x_ref, dego_ref, out_ref):
    xb = x_ref[...]
    d = dego_ref[...]
    scale = jnp.where(d > 0.0, 1.0 / d, 0.0)
    out_ref[0] = xb
    out_ref[1] = xb * scale


def _build_tables(x_pad, dego):
    nb = 10
    bs = NP // nb
    return pl.pallas_call(
        _table_body,
        grid=(nb,),
        in_specs=[
            pl.BlockSpec((bs, F), lambda i: (i, 0)),
            pl.BlockSpec((bs, 1), lambda i: (i, 0)),
        ],
        out_specs=pl.BlockSpec((2, bs, F), lambda i: (0, i, 0)),
        out_shape=jax.ShapeDtypeStruct((2, NP, F), jnp.float32),
    )(x_pad, dego)


NBUF = 2
BPB = 40


@functools.partial(
    pl.kernel,
    out_type=jax.ShapeDtypeStruct((NC, NP, F), jnp.float32),
    mesh=_mesh,
    scratch_types=[
        pltpu.VMEM((BPB, CHUNK), jnp.int32),
        pltpu.VMEM((BPB, CHUNK), jnp.int32),
        pltpu.VMEM((NBUF, CHUNK, F), jnp.float32),
        pltpu.VMEM_SHARED((NP, F), jnp.float32),
        pltpu.SemaphoreType.DMA,
        pltpu.SemaphoreType.DMA,
    ],
)
def _sc_edge_pass(tab_hbm, ri_hbm, ci_hbm, out_hbm, r_v, c_v, g_v, acc_sh,
                  sem0, sem1):
    c = lax.axis_index("c")
    s = lax.axis_index("s")

    def _z(i, _):
        g_v[0, i // 8, pl.ds((i % 8) * 16, 16)] = jnp.zeros((16,), jnp.float32)
        return _
    lax.fori_loop(0, CHUNK * F // 16, _z, 0)
    for j in range(RPN // CHUNK):
        pltpu.sync_copy(g_v.at[0], acc_sh.at[pl.ds(s * RPN + j * CHUNK, CHUNK), :])
    plsc.subcore_barrier()

    sems = (sem0, sem1)

    def _start(k, b):
        pltpu.async_copy(tab_hbm.at[r_v.at[k]], g_v.at[b], sems[b])

    def _wait(k, b):
        pltpu.make_async_copy(tab_hbm.at[r_v.at[k]], g_v.at[b], sems[b]).wait()

    def _block(blk, carry):
        base = s * RPT + blk * BPB
        pltpu.sync_copy(ri_hbm.at[c, pl.ds(base, BPB), :], r_v)
        pltpu.sync_copy(ci_hbm.at[pl.ds(base, BPB), :], c_v)
        for b in range(NBUF):
            _start(b, b)

        def _body(g, inner):
            for b in range(NBUF):
                k = g * NBUF + b
                _wait(k, b)
                pltpu.sync_copy(g_v.at[b], acc_sh.at[c_v.at[k]], add=True)

                @pl.when(k + NBUF < BPB)
                def _go(b=b, k=k):
                    _start(k + NBUF, b)
            return inner

        lax.fori_loop(0, BPB // NBUF, _body, 0)
        return carry

    lax.fori_loop(0, RPT // BPB, _block, 0)
    plsc.subcore_barrier()

    for j in range(RPN // CHUNK):
        r0 = s * RPN + j * CHUNK
        pltpu.sync_copy(acc_sh.at[pl.ds(r0, CHUNK), :], g_v.at[0])
        pltpu.sync_copy(g_v.at[0], out_hbm.at[c, pl.ds(r0, CHUNK), :])


def _dense_body(x_ref, sums_ref, din_ref,
                wz00_ref, wz10_ref, wz01_ref, wz11_ref,
                wh00_ref, wh10_ref, wh01_ref, wh11_ref,
                bz_ref, bh_ref, wl_ref, bl_ref, out_ref):
    xb = x_ref[...]
    S = sums_ref[0]
    O = sums_ref[1]
    din = din_ref[...]
    Ti = S * jnp.where(din > 0.0, 1.0 / din, 0.0)

    dot = functools.partial(jnp.dot, preferred_element_type=jnp.float32)
    zp = (dot(xb, wz00_ref[...] + wz10_ref[...]) + dot(O, wz01_ref[...])
          + dot(Ti, wz11_ref[...]) + bz_ref[...])
    hp = (dot(xb, wh00_ref[...] + wh10_ref[...]) + dot(O, wh01_ref[...])
          + dot(Ti, wh11_ref[...]) + bh_ref[...])
    z = jax.nn.sigmoid(zp)
    ht = jnp.tanh(hp)
    h = jax.nn.relu((1.0 - z) * ht)
    out_ref[...] = dot(h, wl_ref[...]) + bl_ref[...]


def _dense(x_pad, sums, din, Wz, bz, Wh, bh, W_lin, b_lin):
    nb = 10
    bs = NP // nb
    full = lambda shape: pl.BlockSpec(shape, lambda i: tuple(0 for _ in shape))
    return pl.pallas_call(
        _dense_body,
        grid=(nb,),
        in_specs=[
            pl.BlockSpec((bs, F), lambda i: (i, 0)),
            pl.BlockSpec((2, bs, F), lambda i: (0, i, 0)),
            pl.BlockSpec((bs, 1), lambda i: (i, 0)),
            full((F, F)), full((F, F)), full((F, F)), full((F, F)),
            full((F, F)), full((F, F)), full((F, F)), full((F, F)),
            full((1, F)), full((1, F)), full((F, NT)), full((1, NT)),
        ],
        out_specs=pl.BlockSpec((bs, NT), lambda i: (i, 0)),
        out_shape=jax.ShapeDtypeStruct((NP, NT), jnp.float32),
    )(x_pad, sums, din,
      Wz[0, 0, :F], Wz[1, 0, :F], Wz[0, 1, :F], Wz[1, 1, :F],
      Wh[0, 0, :F], Wh[1, 0, :F], Wh[0, 1, :F], Wh[1, 1, :F],
      bz.reshape(1, F), bh.reshape(1, F), W_lin, b_lin.reshape(1, NT))


def kernel(x, edge_index, edge_weight, Wz, bz, Wr, br, Wh, bh, W_lin, b_lin):
    row = edge_index[0].astype(jnp.int32)
    col = edge_index[1].astype(jnp.int32)
    w = edge_weight.astype(jnp.float32)

    npad = EP - E
    pad_idx = (N + (jnp.arange(npad, dtype=jnp.int32) % (NP - N)))
    row_p = jnp.concatenate([row, pad_idx])
    col_p = jnp.concatenate([col, pad_idx])
    w_p = jnp.concatenate([w, jnp.zeros((npad,), jnp.float32)])

    nrows = EP // CHUNK
    ei2 = jnp.stack([row_p, col_p]).reshape(NC, nrows, CHUNK)
    w2 = w_p.reshape(nrows, CHUNK)

    degs = _sc_degrees(ei2, w2)

    x_pad = jnp.concatenate(
        [x.astype(jnp.float32), jnp.zeros((NP - N, F), jnp.float32)])
    tables = _build_tables(x_pad, degs[0].reshape(NP, 1))
    tab_flat = tables.reshape(NC * NP, F)

    ri2 = jnp.stack([row_p, row_p + NP]).reshape(NC, nrows, CHUNK)
    ci2 = col_p.reshape(nrows, CHUNK)
    sums = _sc_edge_pass(tab_flat, ri2, ci2)

    out = _dense(x_pad, sums, degs[1].reshape(NP, 1),
                 Wz, bz, Wh, bh, W_lin, b_lin)
    return out[:N]
